# baseline (speedup 1.0000x reference)
import functools

import jax
import jax.numpy as jnp
from jax import lax
from jax.experimental import pallas as pl
from jax.experimental.pallas import tpu as pltpu
from jax.experimental.pallas import tpu_sc as plsc

BATCH = 16384
EMB = 128
HID = 128
NOUT = 4

NUM_CORES = 2
NUM_SUBCORES = 16
NW = NUM_CORES * NUM_SUBCORES
CHUNK = 128

NSPLIT = 2
HB = BATCH // NSPLIT


def _sc_gather(table, idx2d, batch, row_off):
    b_per_w = batch // NW
    n_chunks = b_per_w // CHUNK
    mesh = plsc.VectorSubcoreMesh(core_axis_name="c", subcore_axis_name="s")

    @functools.partial(
        pl.kernel,
        mesh=mesh,
        out_type=jax.ShapeDtypeStruct((batch, EMB), jnp.float32),
        scratch_types=[
            pltpu.VMEM((n_chunks, CHUNK), jnp.int32),
            pltpu.VMEM((b_per_w, EMB), jnp.float32),
            pltpu.SemaphoreType.DMA,
        ],
    )
    def gather_kernel(table_hbm, idx_hbm, out_hbm, idx_v, rows_v, sem):
        wid = lax.axis_index("s") * NUM_CORES + lax.axis_index("c")
        pltpu.sync_copy(
            idx_hbm.at[pl.ds(row_off + wid * n_chunks, n_chunks)], idx_v)
        copies = []
        for j in range(n_chunks):
            copies.append(
                pltpu.async_copy(
                    table_hbm.at[idx_v.at[j]],
                    rows_v.at[pl.ds(j * CHUNK, CHUNK)],
                    sem,
                )
            )
        for c in copies:
            c.wait()
        pltpu.sync_copy(rows_v, out_hbm.at[pl.ds(wid * b_per_w, b_per_w)])

    return gather_kernel(table, idx2d)


def _mlp_body(x_ref, w1_ref, b1_ref, w2_ref, b2_ref, w3_ref, b3_ref, o_ref):
    x = x_ref[...]
    h = jnp.dot(x, w1_ref[...], preferred_element_type=jnp.float32)
    h = jnp.maximum(h + b1_ref[...], 0.0)
    h = jnp.dot(h, w2_ref[...], preferred_element_type=jnp.float32)
    h = jnp.maximum(h + b2_ref[...], 0.0)
    o = jnp.dot(h, w3_ref[...], preferred_element_type=jnp.float32)
    o_ref[...] = o + b3_ref[...]


def _tc_mlp(emb, W1, b1, W2, b2, W3, b3, batch, blk):
    grid = (batch // blk,)
    return pl.pallas_call(
        _mlp_body,
        grid=grid,
        in_specs=[
            pl.BlockSpec((blk, EMB), lambda i: (i, 0)),
            pl.BlockSpec((EMB, HID), lambda i: (0, 0)),
            pl.BlockSpec((1, HID), lambda i: (0, 0)),
            pl.BlockSpec((HID, HID), lambda i: (0, 0)),
            pl.BlockSpec((1, HID), lambda i: (0, 0)),
            pl.BlockSpec((HID, NOUT), lambda i: (0, 0)),
            pl.BlockSpec((1, NOUT), lambda i: (0, 0)),
        ],
        out_specs=pl.BlockSpec((blk, NOUT), lambda i: (i, 0)),
        out_shape=jax.ShapeDtypeStruct((batch, NOUT), jnp.float32),
    )(emb, W1, b1.reshape(1, HID), W2, b2.reshape(1, HID),
      W3, b3.reshape(1, NOUT))


def kernel(state_idx, table, W1, b1, W2, b2, W3, b3):
    idx2d = state_idx.reshape(BATCH // CHUNK, CHUNK)
    rows_per_half = HB // CHUNK
    outs = []
    for h in range(NSPLIT):
        emb_h = _sc_gather(table, idx2d, HB, h * rows_per_half)
        outs.append(_tc_mlp(emb_h, W1, b1, W2, b2, W3, b3, HB, 2048))
    return jnp.concatenate(outs, axis=0)


# speedup vs baseline: 1.0765x; 1.0765x over previous
import functools

import jax
import jax.numpy as jnp
from jax import lax
from jax.experimental import pallas as pl
from jax.experimental.pallas import tpu as pltpu
from jax.experimental.pallas import tpu_sc as plsc

BATCH = 16384
EMB = 128
HID = 128
NOUT = 4

NUM_CORES = 2
NUM_SUBCORES = 16
NW = NUM_CORES * NUM_SUBCORES
B_PER_W = BATCH // NW
CHUNK = 128
N_CHUNKS = B_PER_W // CHUNK

BLK = 4096


def _sc_gather(table, idx2d):
    mesh = plsc.VectorSubcoreMesh(core_axis_name="c", subcore_axis_name="s")

    @functools.partial(
        pl.kernel,
        mesh=mesh,
        out_type=jax.ShapeDtypeStruct((BATCH, EMB), jnp.float32),
        scratch_types=[
            pltpu.VMEM((N_CHUNKS, CHUNK), jnp.int32),
            pltpu.VMEM((B_PER_W, EMB), jnp.float32),
        ]
        + [pltpu.SemaphoreType.DMA] * N_CHUNKS
        + [pltpu.SemaphoreType.DMA],
    )
    def gather_kernel(table_hbm, idx_hbm, out_hbm, idx_v, rows_v, *sems):
        gsems, ssem = sems[:N_CHUNKS], sems[N_CHUNKS]
        wid = lax.axis_index("s") * NUM_CORES + lax.axis_index("c")
        pltpu.sync_copy(idx_hbm.at[pl.ds(wid * N_CHUNKS, N_CHUNKS)], idx_v)
        gathers = []
        for j in range(N_CHUNKS):
            gathers.append(
                pltpu.async_copy(
                    table_hbm.at[idx_v.at[j]],
                    rows_v.at[pl.ds(j * CHUNK, CHUNK)],
                    gsems[j],
                )
            )
        stores = []
        for j in range(N_CHUNKS):
            gathers[j].wait()
            stores.append(
                pltpu.async_copy(
                    rows_v.at[pl.ds(j * CHUNK, CHUNK)],
                    out_hbm.at[pl.ds(wid * B_PER_W + j * CHUNK, CHUNK)],
                    ssem,
                )
            )
        for s in stores:
            s.wait()

    return gather_kernel(table, idx2d)


def _mlp_body(x_ref, w1_ref, b1_ref, w2_ref, b2_ref, w3_ref, b3_ref, o_ref):
    x = x_ref[...]
    h = jnp.dot(x, w1_ref[...], preferred_element_type=jnp.float32)
    h = jnp.maximum(h + b1_ref[...], 0.0)
    h = jnp.dot(h, w2_ref[...], preferred_element_type=jnp.float32)
    h = jnp.maximum(h + b2_ref[...], 0.0)
    o = jnp.dot(h, w3_ref[...], preferred_element_type=jnp.float32)
    o_ref[...] = o + b3_ref[...]


def _tc_mlp(emb, W1, b1, W2, b2, W3, b3):
    w3_wide = jnp.tile(W3, (1, 128 // NOUT))
    b3_wide = jnp.tile(b3.reshape(1, NOUT), (1, 128 // NOUT))
    grid = (BATCH // BLK,)
    out = pl.pallas_call(
        _mlp_body,
        grid=grid,
        in_specs=[
            pl.BlockSpec((BLK, EMB), lambda i: (i, 0)),
            pl.BlockSpec((EMB, HID), lambda i: (0, 0)),
            pl.BlockSpec((1, HID), lambda i: (0, 0)),
            pl.BlockSpec((HID, HID), lambda i: (0, 0)),
            pl.BlockSpec((1, HID), lambda i: (0, 0)),
            pl.BlockSpec((HID, 128), lambda i: (0, 0)),
            pl.BlockSpec((1, 128), lambda i: (0, 0)),
        ],
        out_specs=pl.BlockSpec((BLK, 128), lambda i: (i, 0)),
        out_shape=jax.ShapeDtypeStruct((BATCH, 128), jnp.float32),
    )(emb, W1, b1.reshape(1, HID), W2, b2.reshape(1, HID),
      w3_wide, b3_wide)
    return lax.slice(out, (0, 0), (BATCH, NOUT))


def kernel(state_idx, table, W1, b1, W2, b2, W3, b3):
    idx2d = state_idx.reshape(NW * N_CHUNKS, CHUNK)
    emb = _sc_gather(table, idx2d)
    return _tc_mlp(emb, W1, b1, W2, b2, W3, b3)
